# Initial kernel scaffold; baseline (speedup 1.0000x reference)
import jax
import jax.numpy as jnp
from jax.experimental import pallas as pl


def kernel(input, weight):
    raise NotImplementedError("write your pallas kernel here")


# speedup vs baseline: 2.9781x; 2.9781x over previous
import functools

import jax
import jax.numpy as jnp
from jax import lax
from jax.experimental import pallas as pl
from jax.experimental.pallas import tpu as pltpu
from jax.experimental.pallas import tpu_sc as plsc

NTOKEN = 100000
NINP = 128
BATCH = 4096
HIST = 50

B = BATCH * HIST
NC, NS = 2, 16
NW = NC * NS
B_PER_W = B // NW
CHUNK = 128
NCHUNK = B_PER_W // CHUNK

_mesh = plsc.VectorSubcoreMesh(core_axis_name="c", subcore_axis_name="s")


@functools.partial(
    pl.kernel,
    mesh=_mesh,
    out_type=jax.ShapeDtypeStruct((B, NINP), jnp.float32),
    scratch_types=[
        pltpu.VMEM((NCHUNK, CHUNK), jnp.int32),
        pltpu.VMEM((CHUNK, NINP), jnp.float32),
        pltpu.SemaphoreType.DMA,
    ],
)
def _embed_lookup(table_hbm, idx_hbm, out_hbm, idx_v, rows_v, sem):
    wid = lax.axis_index("s") * NC + lax.axis_index("c")
    base = wid * B_PER_W
    pltpu.sync_copy(idx_hbm.at[wid], idx_v)

    def body(j, _):
        pltpu.async_copy(table_hbm.at[idx_v.at[j]], rows_v, sem).wait()
        pltpu.sync_copy(rows_v, out_hbm.at[pl.ds(base + j * CHUNK, CHUNK)])
        return 0

    lax.fori_loop(0, NCHUNK, body, 0)


def kernel(input, weight):
    idx = input.astype(jnp.int32).reshape(NW, NCHUNK, CHUNK)
    out = _embed_lookup(weight, idx)
    return out.reshape(BATCH, HIST, NINP)


# speedup vs baseline: 3.3653x; 1.1300x over previous
import functools

import jax
import jax.numpy as jnp
from jax import lax
from jax.experimental import pallas as pl
from jax.experimental.pallas import tpu as pltpu
from jax.experimental.pallas import tpu_sc as plsc

NTOKEN = 100000
NINP = 128
BATCH = 4096
HIST = 50

B = BATCH * HIST
NC, NS = 2, 16
NW = NC * NS
B_PER_W = B // NW
CHUNK = 128
NCHUNK = B_PER_W // CHUNK
NBUF = 5
NOUTER = NCHUNK // NBUF

_mesh = plsc.VectorSubcoreMesh(core_axis_name="c", subcore_axis_name="s")


@functools.partial(
    pl.kernel,
    mesh=_mesh,
    out_type=jax.ShapeDtypeStruct((B, NINP), jnp.float32),
    scratch_types=[
        pltpu.VMEM((NCHUNK, CHUNK), jnp.int32),
        pltpu.VMEM((NBUF, CHUNK, NINP), jnp.float32),
        pltpu.SemaphoreType.DMA((NBUF,)),
    ],
)
def _embed_lookup(table_hbm, idx_hbm, out_hbm, idx_v, rows_v, gsem):
    wid = lax.axis_index("s") * NC + lax.axis_index("c")
    base = wid * B_PER_W
    pltpu.sync_copy(idx_hbm.at[wid], idx_v)

    def gather(j, b):
        pltpu.make_async_copy(
            table_hbm.at[idx_v.at[j]], rows_v.at[b], gsem.at[b]
        ).start()

    def drain(j, b):
        pltpu.make_async_copy(
            table_hbm.at[idx_v.at[j]], rows_v.at[b], gsem.at[b]
        ).wait()

    for b in range(NBUF):
        gather(b, b)

    def body(t, _):
        jbase = t * NBUF
        for b in range(NBUF):
            j = jbase + b
            drain(j, b)
            pltpu.sync_copy(rows_v.at[b], out_hbm.at[pl.ds(base + j * CHUNK, CHUNK)])
            gather(j + NBUF, b)
        return 0

    lax.fori_loop(0, NOUTER - 1, body, 0)

    jbase = (NOUTER - 1) * NBUF
    for b in range(NBUF):
        j = jbase + b
        drain(j, b)
        pltpu.sync_copy(rows_v.at[b], out_hbm.at[pl.ds(base + j * CHUNK, CHUNK)])


def kernel(input, weight):
    idx = input.astype(jnp.int32).reshape(NW, NCHUNK, CHUNK)
    out = _embed_lookup(weight, idx)
    return out.reshape(BATCH, HIST, NINP)


# speedup vs baseline: 6.0238x; 1.7900x over previous
import functools

import jax
import jax.numpy as jnp
from jax import lax
from jax.experimental import pallas as pl
from jax.experimental.pallas import tpu as pltpu
from jax.experimental.pallas import tpu_sc as plsc

NTOKEN = 100000
NINP = 128
BATCH = 4096
HIST = 50

NC, NS = 2, 16
NW = NC * NS
S_PER_W = BATCH // NW
SPC = 2
CHUNK = SPC * HIST
NCHUNK = S_PER_W // SPC
NBUF = 8
NOUTER = NCHUNK // NBUF

_mesh = plsc.VectorSubcoreMesh(core_axis_name="c", subcore_axis_name="s")


@functools.partial(
    pl.kernel,
    mesh=_mesh,
    out_type=jax.ShapeDtypeStruct((BATCH, HIST, NINP), jnp.float32),
    scratch_types=[
        pltpu.VMEM((NCHUNK, CHUNK), jnp.int32),
        pltpu.VMEM((NBUF, CHUNK, NINP), jnp.float32),
        pltpu.SemaphoreType.DMA((NBUF,)),
    ],
)
def _embed_lookup(table_hbm, idx_hbm, out_hbm, idx_v, rows_v, gsem):
    wid = lax.axis_index("s") * NC + lax.axis_index("c")
    sbase = wid * S_PER_W
    pltpu.sync_copy(idx_hbm.at[wid], idx_v)

    def gather(j, b):
        pltpu.make_async_copy(
            table_hbm.at[idx_v.at[j]], rows_v.at[b], gsem.at[b]
        ).start()

    def drain(j, b):
        pltpu.make_async_copy(
            table_hbm.at[idx_v.at[j]], rows_v.at[b], gsem.at[b]
        ).wait()

    def store(j, b):
        s = sbase + j * SPC
        for k in range(SPC):
            pltpu.sync_copy(
                rows_v.at[b, pl.ds(k * HIST, HIST)], out_hbm.at[s + k]
            )

    for b in range(NBUF):
        gather(b, b)

    def body(t, _):
        jbase = t * NBUF
        for b in range(NBUF):
            j = jbase + b
            drain(j, b)
            store(j, b)
            gather(j + NBUF, b)
        return 0

    lax.fori_loop(0, NOUTER - 1, body, 0)

    jbase = (NOUTER - 1) * NBUF
    for b in range(NBUF):
        j = jbase + b
        drain(j, b)
        store(j, b)


def kernel(input, weight):
    idx = input.astype(jnp.int32).reshape(NW, NCHUNK, CHUNK)
    return _embed_lookup(weight, idx)


# speedup vs baseline: 10.7534x; 1.7851x over previous
import functools

import jax
import jax.numpy as jnp
from jax import lax
from jax.experimental import pallas as pl
from jax.experimental.pallas import tpu as pltpu
from jax.experimental.pallas import tpu_sc as plsc

NTOKEN = 100000
NINP = 128
BATCH = 4096
HIST = 50

NC, NS = 2, 16
NW = NC * NS
S_PER_W = BATCH // NW
CHUNK = S_PER_W
NCHUNK = HIST
NBUF = 7

_mesh = plsc.VectorSubcoreMesh(core_axis_name="c", subcore_axis_name="s")


@functools.partial(
    pl.kernel,
    mesh=_mesh,
    out_type=jax.ShapeDtypeStruct((HIST, BATCH, NINP), jnp.float32),
    scratch_types=[
        pltpu.VMEM((NCHUNK, CHUNK), jnp.int32),
        pltpu.VMEM((NBUF, CHUNK, NINP), jnp.float32),
        pltpu.SemaphoreType.DMA((NBUF,)),
    ],
)
def _embed_lookup(table_hbm, idx_hbm, out_hbm, idx_v, rows_v, gsem):
    wid = lax.axis_index("s") * NC + lax.axis_index("c")
    sbase = wid * S_PER_W
    pltpu.sync_copy(idx_hbm.at[wid], idx_v)

    def gather(h, b):
        pltpu.make_async_copy(
            table_hbm.at[idx_v.at[h]], rows_v.at[b], gsem.at[b]
        ).start()

    def drain(h, b):
        pltpu.make_async_copy(
            table_hbm.at[idx_v.at[h]], rows_v.at[b], gsem.at[b]
        ).wait()

    def store(h, b):
        pltpu.sync_copy(rows_v.at[b], out_hbm.at[h, pl.ds(sbase, CHUNK)])

    for b in range(NBUF):
        gather(b, b)

    def body(h, _):
        b = lax.rem(h, NBUF)
        drain(h, b)
        store(h, b)

        @pl.when(h + NBUF < NCHUNK)
        def _():
            gather(h + NBUF, b)

        return 0

    lax.fori_loop(0, NCHUNK, body, 0)


def kernel(input, weight):
    idx = input.astype(jnp.int32).reshape(NW, S_PER_W, HIST).transpose(0, 2, 1)
    out = _embed_lookup(weight, idx)
    return out.transpose(1, 0, 2)


# speedup vs baseline: 10.8071x; 1.0050x over previous
import functools

import jax
import jax.numpy as jnp
from jax import lax
from jax.experimental import pallas as pl
from jax.experimental.pallas import tpu as pltpu
from jax.experimental.pallas import tpu_sc as plsc

NTOKEN = 100000
NINP = 128
BATCH = 4096
HIST = 50

NC, NS = 2, 16
NW = NC * NS
S_PER_W = BATCH // NW
CHUNK = S_PER_W
NCHUNK = HIST
NBUF = 7

_mesh = plsc.VectorSubcoreMesh(core_axis_name="c", subcore_axis_name="s")


@functools.partial(
    pl.kernel,
    mesh=_mesh,
    out_type=jax.ShapeDtypeStruct((HIST, BATCH, NINP), jnp.float32),
    scratch_types=[
        pltpu.VMEM((NCHUNK, CHUNK), jnp.int32),
        pltpu.VMEM((NBUF, CHUNK, NINP), jnp.float32),
        pltpu.SemaphoreType.DMA((NBUF,)),
        pltpu.SemaphoreType.DMA((NBUF,)),
    ],
)
def _embed_lookup(table_hbm, idx_hbm, out_hbm, idx_v, rows_v, gsem, ssem):
    wid = lax.axis_index("s") * NC + lax.axis_index("c")
    sbase = wid * S_PER_W
    pltpu.sync_copy(idx_hbm.at[wid], idx_v)

    def gather(h, b):
        pltpu.make_async_copy(
            table_hbm.at[idx_v.at[h]], rows_v.at[b], gsem.at[b]
        ).start()

    def drain(h, b):
        pltpu.make_async_copy(
            table_hbm.at[idx_v.at[h]], rows_v.at[b], gsem.at[b]
        ).wait()

    def store_start(h, b):
        pltpu.make_async_copy(
            rows_v.at[b], out_hbm.at[h, pl.ds(sbase, CHUNK)], ssem.at[b]
        ).start()

    def store_wait(h, b):
        pltpu.make_async_copy(
            rows_v.at[b], out_hbm.at[h, pl.ds(sbase, CHUNK)], ssem.at[b]
        ).wait()

    for b in range(NBUF):
        gather(b, b)

    def body(h, _):
        b = lax.rem(h, NBUF)
        drain(h, b)
        store_start(h, b)

        @pl.when((h >= 1) & (h + NBUF - 1 < NCHUNK))
        def _():
            b2 = lax.rem(h + NBUF - 1, NBUF)
            store_wait(h - 1, b2)
            gather(h + NBUF - 1, b2)

        return 0

    lax.fori_loop(0, NCHUNK, body, 0)

    for k in range(NBUF):
        h = NCHUNK - NBUF + k
        store_wait(h, h % NBUF)


def kernel(input, weight):
    idx = input.astype(jnp.int32).reshape(NW, S_PER_W, HIST).transpose(0, 2, 1)
    out = _embed_lookup(weight, idx)
    return out.transpose(1, 0, 2)


# speedup vs baseline: 10.8136x; 1.0006x over previous
import functools

import jax
import jax.numpy as jnp
from jax import lax
from jax.experimental import pallas as pl
from jax.experimental.pallas import tpu as pltpu
from jax.experimental.pallas import tpu_sc as plsc

NTOKEN = 100000
NINP = 128
BATCH = 4096
HIST = 50

NC, NS = 2, 16
NW = NC * NS
S_PER_W = BATCH // NW
CHUNK = S_PER_W
NCHUNK = HIST
NBUF = 7

_mesh = plsc.VectorSubcoreMesh(core_axis_name="c", subcore_axis_name="s")


@functools.partial(
    pl.kernel,
    mesh=_mesh,
    compiler_params=pltpu.CompilerParams(
        disable_bounds_checks=True,
        disable_semaphore_checks=True,
        skip_device_barrier=True,
    ),
    out_type=jax.ShapeDtypeStruct((HIST, BATCH, NINP), jnp.float32),
    scratch_types=[
        pltpu.VMEM((NCHUNK, CHUNK), jnp.int32),
        pltpu.VMEM((NBUF, CHUNK, NINP), jnp.float32),
        pltpu.SemaphoreType.DMA((NBUF,)),
        pltpu.SemaphoreType.DMA((NBUF,)),
    ],
)
def _embed_lookup(table_hbm, idx_hbm, out_hbm, idx_v, rows_v, gsem, ssem):
    wid = lax.axis_index("s") * NC + lax.axis_index("c")
    sbase = wid * S_PER_W
    pltpu.sync_copy(idx_hbm.at[wid], idx_v)

    def gather(h, b):
        pltpu.make_async_copy(
            table_hbm.at[idx_v.at[h]], rows_v.at[b], gsem.at[b]
        ).start()

    def drain(h, b):
        pltpu.make_async_copy(
            table_hbm.at[idx_v.at[h]], rows_v.at[b], gsem.at[b]
        ).wait()

    def store_start(h, b):
        pltpu.make_async_copy(
            rows_v.at[b], out_hbm.at[h, pl.ds(sbase, CHUNK)], ssem.at[b]
        ).start()

    def store_wait(h, b):
        pltpu.make_async_copy(
            rows_v.at[b], out_hbm.at[h, pl.ds(sbase, CHUNK)], ssem.at[b]
        ).wait()

    for b in range(NBUF):
        gather(b, b)

    def body(h, _):
        b = lax.rem(h, NBUF)
        drain(h, b)
        store_start(h, b)

        @pl.when((h >= 1) & (h + NBUF - 1 < NCHUNK))
        def _():
            b2 = lax.rem(h + NBUF - 1, NBUF)
            store_wait(h - 1, b2)
            gather(h + NBUF - 1, b2)

        return 0

    lax.fori_loop(0, NCHUNK, body, 0)

    for k in range(NBUF):
        h = NCHUNK - NBUF + k
        store_wait(h, h % NBUF)


def kernel(input, weight):
    idx = input.astype(jnp.int32).reshape(NW, S_PER_W, HIST).transpose(0, 2, 1)
    out = _embed_lookup(weight, idx)
    return out.transpose(1, 0, 2)


# speedup vs baseline: 10.8136x; 1.0000x over previous
import functools

import jax
import jax.numpy as jnp
from jax import lax
from jax.experimental import pallas as pl
from jax.experimental.pallas import tpu as pltpu
from jax.experimental.pallas import tpu_sc as plsc

NTOKEN = 100000
NINP = 128
BATCH = 4096
HIST = 50

_mesh = plsc.VectorSubcoreMesh(core_axis_name="c", subcore_axis_name="s")

NC = _mesh.num_cores
NS = _mesh.num_subcores
NW = NC * NS
S_PER_W = BATCH // NW
CHUNK = S_PER_W
NCHUNK = HIST
NBUF = 7
assert BATCH % NW == 0 and CHUNK <= 128


@functools.partial(
    pl.kernel,
    mesh=_mesh,
    compiler_params=pltpu.CompilerParams(
        disable_bounds_checks=True,
        disable_semaphore_checks=True,
        skip_device_barrier=True,
    ),
    out_type=jax.ShapeDtypeStruct((HIST, BATCH, NINP), jnp.float32),
    scratch_types=[
        pltpu.VMEM((NCHUNK, CHUNK), jnp.int32),
        pltpu.VMEM((NBUF, CHUNK, NINP), jnp.float32),
        pltpu.SemaphoreType.DMA((NBUF,)),
        pltpu.SemaphoreType.DMA((NBUF,)),
    ],
)
def _embed_lookup(table_hbm, idx_hbm, out_hbm, idx_v, rows_v, gsem, ssem):
    wid = lax.axis_index("s") * NC + lax.axis_index("c")
    sbase = wid * S_PER_W
    pltpu.sync_copy(idx_hbm.at[wid], idx_v)

    def gather(h, b):
        pltpu.make_async_copy(
            table_hbm.at[idx_v.at[h]], rows_v.at[b], gsem.at[b]
        ).start()

    def drain(h, b):
        pltpu.make_async_copy(
            table_hbm.at[idx_v.at[h]], rows_v.at[b], gsem.at[b]
        ).wait()

    def store_start(h, b):
        pltpu.make_async_copy(
            rows_v.at[b], out_hbm.at[h, pl.ds(sbase, CHUNK)], ssem.at[b]
        ).start()

    def store_wait(h, b):
        pltpu.make_async_copy(
            rows_v.at[b], out_hbm.at[h, pl.ds(sbase, CHUNK)], ssem.at[b]
        ).wait()

    for b in range(NBUF):
        gather(b, b)

    def body(h, _):
        b = lax.rem(h, NBUF)
        drain(h, b)
        store_start(h, b)

        @pl.when((h >= 1) & (h + NBUF - 1 < NCHUNK))
        def _():
            b2 = lax.rem(h + NBUF - 1, NBUF)
            store_wait(h - 1, b2)
            gather(h + NBUF - 1, b2)

        return 0

    lax.fori_loop(0, NCHUNK, body, 0)

    for k in range(NBUF):
        h = NCHUNK - NBUF + k
        store_wait(h, h % NBUF)


def kernel(input, weight):
    idx = input.astype(jnp.int32).reshape(NW, S_PER_W, HIST).transpose(0, 2, 1)
    out = _embed_lookup(weight, idx)
    return out.transpose(1, 0, 2)
